# baseline (speedup 1.0000x reference)
import functools

import jax
import jax.numpy as jnp
from jax import lax
from jax.experimental import pallas as pl
from jax.experimental.pallas import tpu as pltpu
from jax.experimental.pallas import tpu_sc as plsc

_NUM_CORES = 2
_NUM_SUBCORES = 16
_NW = _NUM_CORES * _NUM_SUBCORES
_LANES = 16


def _tc_widen(table_t):
    D, V = table_t.shape
    VT2 = 1024
    grid = (V + VT2 - 1) // VT2

    def body(t_ref, o_ref):
        o_ref[:, 0:D] = t_ref[...].T

    return pl.pallas_call(
        body,
        grid=(grid,),
        in_specs=[pl.BlockSpec((D, VT2), lambda j: (0, j))],
        out_specs=pl.BlockSpec((VT2, 2 * D), lambda j: (j, 0)),
        out_shape=jax.ShapeDtypeStruct((V, 2 * D), jnp.float32),
        compiler_params=pltpu.CompilerParams(
            dimension_semantics=("parallel",),
        ),
    )(table_t)


def _sc_pool(idx, table):
    L, NW, CH = idx.shape
    V, DP = table.shape
    D = DP // 2
    nchunk = D // _LANES
    inv = jnp.float32(1.0 / L)
    mesh = plsc.VectorSubcoreMesh(core_axis_name="c", subcore_axis_name="s")

    @functools.partial(
        pl.kernel,
        out_type=jax.ShapeDtypeStruct((NW * CH, D), jnp.float32),
        mesh=mesh,
        scratch_types=[
            pltpu.VMEM((L, CH), jnp.int32),
            pltpu.VMEM((CH, DP), jnp.float32),
            pltpu.VMEM((CH, DP), jnp.float32),
            pltpu.VMEM((CH, DP), jnp.float32),
            pltpu.VMEM((CH, DP), jnp.float32),
            pltpu.VMEM((CH, D), jnp.float32),
            pltpu.SemaphoreType.DMA,
            pltpu.SemaphoreType.DMA,
            pltpu.SemaphoreType.DMA,
            pltpu.SemaphoreType.DMA,
        ],
        compiler_params=pltpu.CompilerParams(use_tc_tiling_on_sc=False),
    )
    def k(idx_hbm, table_hbm, out_hbm, idx_v,
          buf0, buf1, buf2, buf3, acc, sem0, sem1, sem2, sem3):
        c = lax.axis_index("c")
        s = lax.axis_index("s")
        w = c * _NUM_SUBCORES + s
        pltpu.sync_copy(idx_hbm.at[:, w], idx_v)

        bufs = (buf0, buf1, buf2, buf3)
        sems = (sem0, sem1, sem2, sem3)
        NB = len(bufs)

        zero = jnp.zeros((_LANES,), jnp.float32)

        @pl.loop(0, CH)
        def _(r):
            for cc in range(nchunk):
                acc[r, pl.ds(cc * _LANES, _LANES)] = zero

        def accum(buf):
            @pl.loop(0, CH, step=4)
            def _(r0):
                for dr in range(4):
                    r = r0 + dr
                    for cc in range(nchunk):
                        sl = pl.ds(cc * _LANES, _LANES)
                        plsc.addupdate(acc.at[r, sl], buf[r, sl])

        for b in range(NB):
            pltpu.async_copy(table_hbm.at[idx_v.at[b]], bufs[b], sems[b])

        @pl.loop(0, L, step=NB)
        def _(p):
            for b in range(NB):
                pltpu.make_async_copy(
                    table_hbm.at[idx_v.at[p + b]], bufs[b], sems[b]
                ).wait()
                accum(bufs[b])

                @pl.when(p + b + NB < L)
                def _():
                    pltpu.async_copy(
                        table_hbm.at[idx_v.at[p + b + NB]], bufs[b], sems[b]
                    )

        @pl.loop(0, CH)
        def _(r):
            for cc in range(nchunk):
                sl = pl.ds(cc * _LANES, _LANES)
                acc[r, sl] = acc[r, sl] * inv

        pltpu.sync_copy(acc, out_hbm.at[pl.ds(w * CH, CH)])

    return k(idx, table)


def _tc_project_t(pooled, Wt, b2):
    B, D = pooled.shape
    _, V = Wt.shape
    VT = 1024
    grid = (V + VT - 1) // VT

    def body(p_ref, w_ref, b_ref, o_ref):
        acc = lax.dot_general(
            w_ref[...].astype(jnp.bfloat16),
            p_ref[...],
            (((0,), (1,)), ((), ())),
            preferred_element_type=jnp.float32,
        )
        o_ref[...] = acc + b_ref[...].reshape(VT, 1)

    return pl.pallas_call(
        body,
        grid=(grid,),
        in_specs=[
            pl.BlockSpec((B, D), lambda j: (0, 0)),
            pl.BlockSpec((D, VT), lambda j: (0, j)),
            pl.BlockSpec((1, VT), lambda j: (0, j)),
        ],
        out_specs=pl.BlockSpec((VT, B), lambda j: (j, 0)),
        out_shape=jax.ShapeDtypeStruct((V, B), jnp.float32),
        compiler_params=pltpu.CompilerParams(
            dimension_semantics=("parallel",),
        ),
    )(pooled, Wt, b2)


def kernel(context, emb_table, W, b):
    B, L = context.shape
    V, D = emb_table.shape
    CH = B // _NW
    idx = context.T.reshape(L, _NW, CH)
    pooled = _sc_pool(idx, _tc_widen(emb_table.T))
    out_t = _tc_project_t(pooled.astype(jnp.bfloat16), W.T, b.reshape(1, V))
    return out_t.T


# speedup vs baseline: 1.0626x; 1.0626x over previous
import functools

import jax
import jax.numpy as jnp
from jax import lax
from jax.experimental import pallas as pl
from jax.experimental.pallas import tpu as pltpu
from jax.experimental.pallas import tpu_sc as plsc

_NUM_CORES = 2
_NUM_SUBCORES = 16
_NW = _NUM_CORES * _NUM_SUBCORES
_LANES = 16


def _sc_pool(idx, table):
    L, NW, CH = idx.shape
    V, D = table.shape
    nchunk = D // _LANES
    inv = jnp.float32(1.0 / L)
    mesh = plsc.VectorSubcoreMesh(core_axis_name="c", subcore_axis_name="s")

    @functools.partial(
        pl.kernel,
        out_type=jax.ShapeDtypeStruct((NW * CH, D), jnp.float32),
        mesh=mesh,
        scratch_types=[
            pltpu.VMEM((L, CH), jnp.int32),
            pltpu.VMEM((CH, D), jnp.float32),
            pltpu.VMEM((CH, D), jnp.float32),
            pltpu.VMEM((CH, D), jnp.float32),
            pltpu.VMEM((CH, D), jnp.float32),
            pltpu.VMEM((CH, D), jnp.float32),
            pltpu.SemaphoreType.DMA,
            pltpu.SemaphoreType.DMA,
            pltpu.SemaphoreType.DMA,
            pltpu.SemaphoreType.DMA,
        ],
        compiler_params=pltpu.CompilerParams(use_tc_tiling_on_sc=False),
    )
    def k(idx_hbm, table_hbm, out_hbm, idx_v,
          buf0, buf1, buf2, buf3, acc, sem0, sem1, sem2, sem3):
        c = lax.axis_index("c")
        s = lax.axis_index("s")
        w = c * _NUM_SUBCORES + s
        pltpu.sync_copy(idx_hbm.at[:, w], idx_v)

        bufs = (buf0, buf1, buf2, buf3)
        sems = (sem0, sem1, sem2, sem3)
        NB = len(bufs)

        zero = jnp.zeros((_LANES,), jnp.float32)

        @pl.loop(0, CH)
        def _(r):
            for cc in range(nchunk):
                acc[r, pl.ds(cc * _LANES, _LANES)] = zero

        def accum(buf):
            @pl.loop(0, CH, step=4)
            def _(r0):
                for dr in range(4):
                    r = r0 + dr
                    for cc in range(nchunk):
                        sl = pl.ds(cc * _LANES, _LANES)
                        plsc.addupdate(acc.at[r, sl], buf[r, sl])

        for b in range(NB):
            pltpu.async_copy(table_hbm.at[idx_v.at[b]], bufs[b], sems[b])

        @pl.loop(0, L, step=NB)
        def _(p):
            for b in range(NB):
                pltpu.make_async_copy(
                    table_hbm.at[idx_v.at[p + b]], bufs[b], sems[b]
                ).wait()
                accum(bufs[b])

                @pl.when(p + b + NB < L)
                def _():
                    pltpu.async_copy(
                        table_hbm.at[idx_v.at[p + b + NB]], bufs[b], sems[b]
                    )

        @pl.loop(0, CH)
        def _(r):
            for cc in range(nchunk):
                sl = pl.ds(cc * _LANES, _LANES)
                acc[r, sl] = acc[r, sl] * inv

        pltpu.sync_copy(acc, out_hbm.at[pl.ds(w * CH, CH)])

    return k(idx, table)


def _tc_project_t(pooled, Wt, b2):
    B, D = pooled.shape
    _, V = Wt.shape
    VT = 1024
    grid = (V + VT - 1) // VT

    def body(p_ref, w_ref, b_ref, o_ref):
        acc = lax.dot_general(
            w_ref[...].astype(jnp.bfloat16),
            p_ref[...],
            (((0,), (1,)), ((), ())),
            preferred_element_type=jnp.float32,
        )
        o_ref[...] = acc + b_ref[...].reshape(VT, 1)

    return pl.pallas_call(
        body,
        grid=(grid,),
        in_specs=[
            pl.BlockSpec((B, D), lambda j: (0, 0)),
            pl.BlockSpec((D, VT), lambda j: (0, j)),
            pl.BlockSpec((1, VT), lambda j: (0, j)),
        ],
        out_specs=pl.BlockSpec((VT, B), lambda j: (j, 0)),
        out_shape=jax.ShapeDtypeStruct((V, B), jnp.float32),
        compiler_params=pltpu.CompilerParams(
            dimension_semantics=("parallel",),
        ),
    )(pooled, Wt, b2)


def kernel(context, emb_table, W, b):
    B, L = context.shape
    V, D = emb_table.shape
    CH = B // _NW
    idx = context.T.reshape(L, _NW, CH)
    pooled = _sc_pool(idx, emb_table)
    out_t = _tc_project_t(pooled.astype(jnp.bfloat16), W.T, b.reshape(1, V))
    return out_t.T
